# baseline (speedup 1.0000x reference)
import functools

import jax
import jax.numpy as jnp
from jax import lax
from jax.experimental import pallas as pl
from jax.experimental.pallas import tpu as pltpu
from jax.experimental.pallas import tpu_sc as plsc

N = 10000
E = 160000
EP = E + N
IN_DIM = 128
HID = 128
OUT_DIM = 256
HEADS = 4
HEAD_DIM = 32

N_PAD = 10240
TR = 256


def _mm_body(x_ref, w_ref, o_ref):
    o_ref[...] = jnp.dot(x_ref[...], w_ref[...],
                         preferred_element_type=jnp.float32)


def _mm(x, w):
    r, k = x.shape[0], w.shape[1]
    return pl.pallas_call(
        _mm_body,
        grid=(r // TR,),
        in_specs=[pl.BlockSpec((TR, x.shape[1]), lambda i: (i, 0)),
                  pl.BlockSpec((x.shape[1], k), lambda i: (0, 0))],
        out_specs=pl.BlockSpec((TR, k), lambda i: (i, 0)),
        out_shape=jax.ShapeDtypeStruct((r, k), jnp.float32),
    )(x, w)


def _stageb_body(agg_ref, den_ref, b_ref, w_ref, o_ref):
    a = agg_ref[0, 0] + agg_ref[0, 1]
    d = (den_ref[0, 0] + den_ref[0, 1])[:, 0:4]
    drep = jnp.broadcast_to(d[:, :, None], (TR, 4, 32)).reshape(TR, 128)
    t = a / (drep + 1e-16) + b_ref[0]
    t = jnp.where(t > 0, t, jnp.exp(t) - 1.0)
    o_ref[0] = jnp.dot(t, w_ref[0], preferred_element_type=jnp.float32)


def _stageb(agg, den, b, w):
    k = w.shape[2]
    return pl.pallas_call(
        _stageb_body,
        grid=(2, N_PAD // TR),
        in_specs=[pl.BlockSpec((1, 2, TR, 128), lambda g, i: (g, 0, i, 0)),
                  pl.BlockSpec((1, 2, TR, 16), lambda g, i: (g, 0, i, 0)),
                  pl.BlockSpec((1, 1, 128), lambda g, i: (g, 0, 0)),
                  pl.BlockSpec((1, 128, k), lambda g, i: (g, 0, 0))],
        out_specs=pl.BlockSpec((1, TR, k), lambda g, i: (g, i, 0)),
        out_shape=jax.ShapeDtypeStruct((2, N_PAD, k), jnp.float32),
    )(agg, den, b[:, None, :], w)


def _stagec_body(af_ref, ar_ref, df_ref, dr_ref, b2f_ref, b2r_ref,
                 w1_ref, b1_ref, w2_ref, b2_ref, g_ref, be_ref, o_ref):
    af = af_ref[0] + af_ref[1]
    ar = ar_ref[0] + ar_ref[1]
    df = (df_ref[0] + df_ref[1])[:, 0:1]
    dr = (dr_ref[0] + dr_ref[1])[:, 0:1]
    hf = af / (df + 1e-16) + b2f_ref[...]
    hr = ar / (dr + 1e-16) + b2r_ref[...]
    h = jnp.concatenate([hf, hr], axis=-1)
    h = jnp.dot(h, w1_ref[...], preferred_element_type=jnp.float32) + b1_ref[...]
    h = 0.5 * h * (1.0 + lax.erf(h / jnp.sqrt(2.0).astype(jnp.float32)))
    h = jnp.dot(h, w2_ref[...], preferred_element_type=jnp.float32) + b2_ref[...]
    mu = jnp.mean(h, axis=-1, keepdims=True)
    var = jnp.mean((h - mu) ** 2, axis=-1, keepdims=True)
    o_ref[...] = (h - mu) / jnp.sqrt(var + 1e-5) * g_ref[...] + be_ref[...]


def _stagec(aggf, aggr, denf, denr, b2f, b2r,
            wfu1, bfu1, wfu2, bfu2, gamma, beta):
    vec = lambda d: pl.BlockSpec((1, d), lambda i: (0, 0))
    return pl.pallas_call(
        _stagec_body,
        grid=(N_PAD // TR,),
        in_specs=[pl.BlockSpec((2, TR, 128), lambda i: (0, i, 0)),
                  pl.BlockSpec((2, TR, 128), lambda i: (0, i, 0)),
                  pl.BlockSpec((2, TR, 16), lambda i: (0, i, 0)),
                  pl.BlockSpec((2, TR, 16), lambda i: (0, i, 0)),
                  vec(128), vec(128),
                  pl.BlockSpec((256, 256), lambda i: (0, 0)), vec(256),
                  pl.BlockSpec((256, 256), lambda i: (0, 0)), vec(256),
                  vec(256), vec(256)],
        out_specs=pl.BlockSpec((TR, 256), lambda i: (i, 0)),
        out_shape=jax.ShapeDtypeStruct((N_PAD, 256), jnp.float32),
    )(aggf, aggr, denf, denr, b2f[None], b2r[None], wfu1, bfu1[None],
      wfu2, bfu2[None], gamma[None], beta[None])


EP_PAD = 172032
NT = 16
NW = 2 * NT
CD = 48
CN = 64
CHD = EP_PAD // NW // CD
CHN = EP_PAD // NW // CN
N_ROWS = 10112
ROWS_PT = N_ROWS // NT
NH = N_ROWS // 2
NHR = 5120
HROWS_PT = NHR // NT


_SC_MESH = dict(core_axis_name="c", subcore_axis_name="s", num_cores=2)


def _make_den_sc():
    @functools.partial(
        pl.kernel,
        out_type=(jax.ShapeDtypeStruct((EP_PAD, 16), jnp.float32),
                  jax.ShapeDtypeStruct((2, 2, NHR, 128), jnp.float32)),
        mesh=plsc.VectorSubcoreMesh(**_SC_MESH),
        scratch_types=[
            pltpu.VMEM((CD,), jnp.int32),
            pltpu.VMEM((1, CD), jnp.int32),
            pltpu.VMEM((1, CD), jnp.int32),
            pltpu.VMEM((1, CD), jnp.int32),
            pltpu.VMEM((CD, 128), jnp.float32),
            pltpu.VMEM((CD, 128), jnp.float32),
            pltpu.VMEM((CD, 16), jnp.float32),
            pltpu.VMEM((CD, 128), jnp.float32),
            pltpu.VMEM((8, 128), jnp.float32),
            pltpu.VMEM_SHARED((NHR, 128), jnp.float32),
            pltpu.SemaphoreType.DMA,
        ])
    def den(pk, astab, adtab, exout, dent,
            pk_v, idx_s, idx_dg, idx_dl, as_v, ad_v, ex16_v, exb_v, z_v,
            den_sp, sem):
        s = lax.axis_index("s")
        c = lax.axis_index("c")
        w = c * NT + s
        zero16 = jnp.zeros((16,), jnp.float32)
        for i in range(8):
            for j in range(8):
                z_v[i, pl.ds(16 * j, 16)] = zero16

        def zexb(i, cry):
            for j in range(1, 8):
                exb_v[i, pl.ds(16 * j, 16)] = zero16
            return cry
        lax.fori_loop(0, CD, zexb, 0)

        def zsp(k, carry):
            pltpu.sync_copy(z_v, den_sp.at[pl.ds(s * HROWS_PT + k * 8, 8)])
            return carry

        def unpack(g, half):
            pltpu.sync_copy(pk.at[pl.ds(g * CD, CD)], pk_v)
            base = half * NH
            for t in range(CD // 16):
                p = pk_v[pl.ds(16 * t, 16)]
                d = p >> 15
                dl = d - base
                ok = (dl >= 0) & (dl < NH)
                idx_s[0, pl.ds(16 * t, 16)] = p & 32767
                idx_dg[0, pl.ds(16 * t, 16)] = d
                idx_dl[0, pl.ds(16 * t, 16)] = jnp.where(ok, dl, NHR - 1)

        lax.fori_loop(0, HROWS_PT // 8, zsp, 0)
        plsc.subcore_barrier()

        def chunk0(ci, carry):
            g = w * CHD + ci
            unpack(g, 0)
            cps = [pltpu.async_copy(astab.at[idx_s.at[0]], as_v, sem),
                   pltpu.async_copy(adtab.at[idx_dg.at[0]], ad_v, sem)]
            for cp in cps:
                cp.wait()

            def rowf(i, cry):
                e = as_v[i, pl.ds(0, 16)] + ad_v[i, pl.ds(0, 16)]
                e = jnp.where(e > 0, e, 0.2 * e)
                ex = jnp.exp(e)
                ex16_v[i, :] = ex
                exb_v[i, pl.ds(0, 16)] = ex
                return cry
            lax.fori_loop(0, CD, rowf, 0)
            pltpu.sync_copy(ex16_v, exout.at[pl.ds(g * CD, CD)])
            pltpu.sync_copy(exb_v, den_sp.at[idx_dl.at[0]], add=True)
            return carry
        lax.fori_loop(0, CHD, chunk0, 0)
        plsc.subcore_barrier()
        rows = pl.ds(s * HROWS_PT, HROWS_PT)
        pltpu.sync_copy(den_sp.at[rows], dent.at[c, 0, rows])
        plsc.subcore_barrier()

        lax.fori_loop(0, HROWS_PT // 8, zsp, 0)
        plsc.subcore_barrier()

        def chunk1(ci, carry):
            g = w * CHD + ci
            unpack(g, 1)
            pltpu.sync_copy(exout.at[pl.ds(g * CD, CD)], ex16_v)

            def cpf(i, cry):
                exb_v[i, pl.ds(0, 16)] = ex16_v[i, :]
                return cry
            lax.fori_loop(0, CD, cpf, 0)
            pltpu.sync_copy(exb_v, den_sp.at[idx_dl.at[0]], add=True)
            return carry
        lax.fori_loop(0, CHD, chunk1, 0)
        plsc.subcore_barrier()
        pltpu.sync_copy(den_sp.at[rows], dent.at[c, 1, rows])

    return den


def _make_num_sc(half, hd_vregs):
    @functools.partial(
        pl.kernel,
        out_type=jax.ShapeDtypeStruct((2, NHR, 128), jnp.float32),
        mesh=plsc.VectorSubcoreMesh(**_SC_MESH),
        scratch_types=[
            pltpu.VMEM((CN,), jnp.int32),
            pltpu.VMEM((1, CN), jnp.int32),
            pltpu.VMEM((1, CN), jnp.int32),
            pltpu.VMEM((1, CN), jnp.int32),
            pltpu.VMEM((1, CN), jnp.int32),
            pltpu.VMEM((CN, 16), jnp.float32),
            pltpu.VMEM((CN, 16), jnp.float32),
            pltpu.VMEM((CN, 128), jnp.float32),
            pltpu.VMEM((CN, 128), jnp.float32),
            pltpu.VMEM((8, 128), jnp.float32),
            pltpu.VMEM_SHARED((NHR, 128), jnp.float32),
            pltpu.SemaphoreType.DMA,
            pltpu.SemaphoreType.DMA,
        ])
    def num(pk, exin, htab, aggh,
            pk_v, idx_s0, idx_s1, idx_dl0, idx_dl1, ex0, ex1, h0, h1,
            z_v, out_sp, sem0, sem1):
        s = lax.axis_index("s")
        c = lax.axis_index("c")
        w = c * NT + s
        idx_s = (idx_s0, idx_s1)
        idx_dl = (idx_dl0, idx_dl1)
        ex16 = (ex0, ex1)
        h_v = (h0, h1)
        sems = (sem0, sem1)
        zero16 = jnp.zeros((16,), jnp.float32)
        for i in range(8):
            for j in range(8):
                z_v[i, pl.ds(16 * j, 16)] = zero16

        def zsp(k, carry):
            pltpu.sync_copy(z_v, out_sp.at[pl.ds(s * HROWS_PT + k * 8, 8)])
            return carry
        lax.fori_loop(0, HROWS_PT // 8, zsp, 0)
        plsc.subcore_barrier()

        def issue(g, b):
            pltpu.sync_copy(pk.at[pl.ds(g * CN, CN)], pk_v)
            for t in range(CN // 16):
                p = pk_v[pl.ds(16 * t, 16)]
                dl = (p >> 15) - half * NH
                ok = (dl >= 0) & (dl < NH)
                idx_s[b][0, pl.ds(16 * t, 16)] = p & 32767
                idx_dl[b][0, pl.ds(16 * t, 16)] = jnp.where(ok, dl, NHR - 1)
            pltpu.async_copy(exin.at[pl.ds(g * CN, CN)], ex16[b], sems[b])
            pltpu.async_copy(htab.at[idx_s[b].at[0]], h_v[b], sems[b])

        def consume(b):
            pltpu.make_async_copy(exin.at[pl.ds(0, CN)], ex16[b],
                                  sems[b]).wait()
            pltpu.make_async_copy(htab.at[pl.ds(0, CN)], h_v[b],
                                  sems[b]).wait()

            def mulf(i, cry):
                av = ex16[b][i, :]
                for j in range(8):
                    a = av[j // hd_vregs]
                    h_v[b][i, pl.ds(16 * j, 16)] = (
                        h_v[b][i, pl.ds(16 * j, 16)] * a)
                return cry
            lax.fori_loop(0, CN, mulf, 0)
            pltpu.sync_copy(h_v[b], out_sp.at[idx_dl[b].at[0]], add=True)

        base = w * CHN
        issue(base, 0)

        def pair(i, carry):
            consume(0)
            issue(base + 2 * i + 1, 1)
            consume(1)

            @pl.when(i < CHN // 2 - 1)
            def _():
                issue(base + 2 * i + 2, 0)
            return carry
        lax.fori_loop(0, CHN // 2, pair, 0)
        plsc.subcore_barrier()
        rows = pl.ds(s * HROWS_PT, HROWS_PT)
        pltpu.sync_copy(out_sp.at[rows], aggh.at[c, rows])

    return num


_den_sc = _make_den_sc()
_num_sc = {(h, hd): _make_num_sc(h, hd)
           for h in (0, 1) for hd in (HEAD_DIM // 16, HID // 16)}


def _gat_layer_sc(pk, astab, adtab, htab, hd_vregs):
    ex, dent = _den_sc(pk, astab, adtab)
    lo = _num_sc[(0, hd_vregs)](pk, ex, htab)
    hi = _num_sc[(1, hd_vregs)](pk, ex, htab)
    agg = jnp.concatenate([lo[:, :NH], hi[:, :NH]], axis=1)
    den = jnp.concatenate([dent[:, 0, :NH, 0:16], dent[:, 1, :NH, 0:16]],
                          axis=1)
    return agg, den


def _edge_prep(ei):
    loops = jnp.arange(N, dtype=jnp.int32)
    src = jnp.concatenate([ei[0], loops])
    dst = jnp.concatenate([ei[1], loops])
    src = jnp.pad(src, (0, EP_PAD - EP), constant_values=N)
    dst = jnp.pad(dst, (0, EP_PAD - EP), constant_values=N)
    return src, dst


def _fold_att(W, att):
    heads, hd = att.shape
    Wr = W.reshape(W.shape[0], heads, hd)
    return jnp.einsum('ihd,hd->ih', Wr, att)


def kernel(x, form_edge_index, role_edge_index,
           W1f, as1f, ad1f, b1f, W2f, as2f, ad2f, b2f,
           W1r, as1r, ad1r, b1r, W2r, as2r, ad2r, b2r,
           Wfu1, bfu1, Wfu2, bfu2, gamma, beta):
    f32 = jnp.float32
    fs, fd = _edge_prep(form_edge_index)
    rs, rd = _edge_prep(role_edge_index)
    pk_f = fs | (fd << 15)
    pk_r = rs | (rd << 15)

    xp = jnp.pad(x, ((0, N_PAD - N), (0, 0)))
    att_cols = jnp.concatenate(
        [_fold_att(W1f, as1f), _fold_att(W1f, ad1f),
         _fold_att(W1r, as1r), _fold_att(W1r, ad1r)], axis=1)
    wcat = jnp.concatenate(
        [W1f, W1r, jnp.pad(att_cols, ((0, 0), (0, 112)))], axis=1)
    H = _mm(xp, wcat)

    p128 = lambda a: jnp.pad(a[:N_ROWS], ((0, 0), (0, 128 - a.shape[1])))
    pn = lambda a: jnp.pad(a, ((0, 0), (0, N_PAD - N_ROWS), (0, 0)))
    a1f, d1f = _gat_layer_sc(pk_f, p128(H[:, 256:260]), p128(H[:, 260:264]),
                             H[:N_ROWS, 0:128], HEAD_DIM // 16)
    a1r, d1r = _gat_layer_sc(pk_r, p128(H[:, 264:268]), p128(H[:, 268:272]),
                             H[:N_ROWS, 128:256], HEAD_DIM // 16)
    agg1 = jnp.stack([pn(a1f), pn(a1r)])
    den1 = jnp.stack([pn(d1f), pn(d1r)])

    b1 = jnp.stack([b1f, b1r])
    w2cat = jnp.stack([
        jnp.concatenate([W2f, _fold_att(W2f, as2f), _fold_att(W2f, ad2f),
                         jnp.zeros((128, 126), f32)], axis=1),
        jnp.concatenate([W2r, _fold_att(W2r, as2r), _fold_att(W2r, ad2r),
                         jnp.zeros((128, 126), f32)], axis=1)])
    H2 = _stageb(agg1, den1, b1, w2cat)

    a2f, d2f = _gat_layer_sc(pk_f, p128(H2[0, :, 128:129]),
                             p128(H2[0, :, 129:130]),
                             H2[0, :N_ROWS, 0:128], HID // 16)
    a2r, d2r = _gat_layer_sc(pk_r, p128(H2[1, :, 128:129]),
                             p128(H2[1, :, 129:130]),
                             H2[1, :N_ROWS, 0:128], HID // 16)

    out = _stagec(pn(a2f), pn(a2r), pn(d2f), pn(d2r),
                  b2f, b2r, Wfu1, bfu1, Wfu2, bfu2, gamma, beta)
    return out[:N]


# speedup vs baseline: 1.0624x; 1.0624x over previous
import functools

import jax
import jax.numpy as jnp
from jax import lax
from jax.experimental import pallas as pl
from jax.experimental.pallas import tpu as pltpu
from jax.experimental.pallas import tpu_sc as plsc

N = 10000
E = 160000
EP = E + N
IN_DIM = 128
HID = 128
OUT_DIM = 256
HEADS = 4
HEAD_DIM = 32

N_PAD = 10240
TR = 256


def _mm_body(x_ref, w_ref, o_ref):
    o_ref[...] = jnp.dot(x_ref[...], w_ref[...],
                         preferred_element_type=jnp.float32)


def _mm(x, w):
    r, k = x.shape[0], w.shape[1]
    return pl.pallas_call(
        _mm_body,
        grid=(r // TR,),
        in_specs=[pl.BlockSpec((TR, x.shape[1]), lambda i: (i, 0)),
                  pl.BlockSpec((x.shape[1], k), lambda i: (0, 0))],
        out_specs=pl.BlockSpec((TR, k), lambda i: (i, 0)),
        out_shape=jax.ShapeDtypeStruct((r, k), jnp.float32),
    )(x, w)


def _stageb_body(agg_ref, den_ref, b_ref, w_ref, o_ref):
    a = agg_ref[0, 0] + agg_ref[0, 1]
    d = (den_ref[0, 0] + den_ref[0, 1])[:, 0:4]
    drep = jnp.broadcast_to(d[:, :, None], (TR, 4, 32)).reshape(TR, 128)
    t = a / (drep + 1e-16) + b_ref[0]
    t = jnp.where(t > 0, t, jnp.exp(t) - 1.0)
    o_ref[0] = jnp.dot(t, w_ref[0], preferred_element_type=jnp.float32)


def _stageb(agg, den, b, w):
    k = w.shape[2]
    return pl.pallas_call(
        _stageb_body,
        grid=(2, N_PAD // TR),
        in_specs=[pl.BlockSpec((1, 2, TR, 128), lambda g, i: (g, 0, i, 0)),
                  pl.BlockSpec((1, 2, TR, 16), lambda g, i: (g, 0, i, 0)),
                  pl.BlockSpec((1, 1, 128), lambda g, i: (g, 0, 0)),
                  pl.BlockSpec((1, 128, k), lambda g, i: (g, 0, 0))],
        out_specs=pl.BlockSpec((1, TR, k), lambda g, i: (g, i, 0)),
        out_shape=jax.ShapeDtypeStruct((2, N_PAD, k), jnp.float32),
    )(agg, den, b[:, None, :], w)


def _stagec_body(af_ref, ar_ref, df_ref, dr_ref, b2f_ref, b2r_ref,
                 w1_ref, b1_ref, w2_ref, b2_ref, g_ref, be_ref, o_ref):
    af = af_ref[0] + af_ref[1]
    ar = ar_ref[0] + ar_ref[1]
    df = (df_ref[0] + df_ref[1])[:, 0:1]
    dr = (dr_ref[0] + dr_ref[1])[:, 0:1]
    hf = af / (df + 1e-16) + b2f_ref[...]
    hr = ar / (dr + 1e-16) + b2r_ref[...]
    h = jnp.concatenate([hf, hr], axis=-1)
    h = jnp.dot(h, w1_ref[...], preferred_element_type=jnp.float32) + b1_ref[...]
    h = 0.5 * h * (1.0 + lax.erf(h / jnp.sqrt(2.0).astype(jnp.float32)))
    h = jnp.dot(h, w2_ref[...], preferred_element_type=jnp.float32) + b2_ref[...]
    mu = jnp.mean(h, axis=-1, keepdims=True)
    var = jnp.mean((h - mu) ** 2, axis=-1, keepdims=True)
    o_ref[...] = (h - mu) / jnp.sqrt(var + 1e-5) * g_ref[...] + be_ref[...]


def _stagec(aggf, aggr, denf, denr, b2f, b2r,
            wfu1, bfu1, wfu2, bfu2, gamma, beta):
    vec = lambda d: pl.BlockSpec((1, d), lambda i: (0, 0))
    return pl.pallas_call(
        _stagec_body,
        grid=(N_PAD // TR,),
        in_specs=[pl.BlockSpec((2, TR, 128), lambda i: (0, i, 0)),
                  pl.BlockSpec((2, TR, 128), lambda i: (0, i, 0)),
                  pl.BlockSpec((2, TR, 16), lambda i: (0, i, 0)),
                  pl.BlockSpec((2, TR, 16), lambda i: (0, i, 0)),
                  vec(128), vec(128),
                  pl.BlockSpec((256, 256), lambda i: (0, 0)), vec(256),
                  pl.BlockSpec((256, 256), lambda i: (0, 0)), vec(256),
                  vec(256), vec(256)],
        out_specs=pl.BlockSpec((TR, 256), lambda i: (i, 0)),
        out_shape=jax.ShapeDtypeStruct((N_PAD, 256), jnp.float32),
    )(aggf, aggr, denf, denr, b2f[None], b2r[None], wfu1, bfu1[None],
      wfu2, bfu2[None], gamma[None], beta[None])


EP_PAD = 172032
NT = 16
NW = 2 * NT
CD = 64
CN = 128
CHD = EP_PAD // NW // CD
CHN = EP_PAD // NW // CN
N_ROWS = 10112
ROWS_PT = N_ROWS // NT
NH = N_ROWS // 2
NHR = 5120
HROWS_PT = NHR // NT


_SC_MESH = dict(core_axis_name="c", subcore_axis_name="s", num_cores=2)


def _make_den_sc():
    @functools.partial(
        pl.kernel,
        out_type=(jax.ShapeDtypeStruct((EP_PAD, 16), jnp.float32),
                  jax.ShapeDtypeStruct((2, 2, NHR, 128), jnp.float32)),
        mesh=plsc.VectorSubcoreMesh(**_SC_MESH),
        scratch_types=[
            pltpu.VMEM((CD,), jnp.int32),
            pltpu.VMEM((1, CD), jnp.int32),
            pltpu.VMEM((1, CD), jnp.int32),
            pltpu.VMEM((1, CD), jnp.int32),
            pltpu.VMEM((CD, 128), jnp.float32),
            pltpu.VMEM((CD, 128), jnp.float32),
            pltpu.VMEM((CD, 16), jnp.float32),
            pltpu.VMEM((8, 128), jnp.float32),
            pltpu.VMEM_SHARED((NHR, 128), jnp.float32),
            pltpu.SemaphoreType.DMA,
        ])
    def den(pk, astab, adtab, exout, dent,
            pk_v, idx_s, idx_dg, idx_dl, as_v, ad_v, ex16_v, z_v,
            den_sp, sem):
        exb_v = as_v
        s = lax.axis_index("s")
        c = lax.axis_index("c")
        w = c * NT + s
        zero16 = jnp.zeros((16,), jnp.float32)
        for i in range(8):
            for j in range(8):
                z_v[i, pl.ds(16 * j, 16)] = zero16


        def zsp(k, carry):
            pltpu.sync_copy(z_v, den_sp.at[pl.ds(s * HROWS_PT + k * 8, 8)])
            return carry

        def unpack(g, half):
            pltpu.sync_copy(pk.at[pl.ds(g * CD, CD)], pk_v)
            base = half * NH
            for t in range(CD // 16):
                p = pk_v[pl.ds(16 * t, 16)]
                d = p >> 15
                dl = d - base
                ok = (dl >= 0) & (dl < NH)
                idx_s[0, pl.ds(16 * t, 16)] = p & 32767
                idx_dg[0, pl.ds(16 * t, 16)] = d
                idx_dl[0, pl.ds(16 * t, 16)] = jnp.where(ok, dl, NHR - 1)

        lax.fori_loop(0, HROWS_PT // 8, zsp, 0)
        plsc.subcore_barrier()

        def chunk0(ci, carry):
            g = w * CHD + ci
            unpack(g, 0)
            cps = [pltpu.async_copy(astab.at[idx_s.at[0]], as_v, sem),
                   pltpu.async_copy(adtab.at[idx_dg.at[0]], ad_v, sem)]
            for cp in cps:
                cp.wait()

            def rowf(i, cry):
                e = as_v[i, pl.ds(0, 16)] + ad_v[i, pl.ds(0, 16)]
                e = jnp.where(e > 0, e, 0.2 * e)
                ex = jnp.exp(e)
                ex16_v[i, :] = ex
                exb_v[i, pl.ds(0, 16)] = ex
                for j in range(1, 8):
                    exb_v[i, pl.ds(16 * j, 16)] = zero16
                return cry
            lax.fori_loop(0, CD, rowf, 0)
            pltpu.sync_copy(ex16_v, exout.at[pl.ds(g * CD, CD)])
            pltpu.sync_copy(exb_v, den_sp.at[idx_dl.at[0]], add=True)
            return carry
        lax.fori_loop(0, CHD, chunk0, 0)
        plsc.subcore_barrier()
        rows = pl.ds(s * HROWS_PT, HROWS_PT)
        pltpu.sync_copy(den_sp.at[rows], dent.at[c, 0, rows])
        plsc.subcore_barrier()

        lax.fori_loop(0, HROWS_PT // 8, zsp, 0)
        plsc.subcore_barrier()

        def chunk1(ci, carry):
            g = w * CHD + ci
            unpack(g, 1)
            pltpu.sync_copy(exout.at[pl.ds(g * CD, CD)], ex16_v)

            def cpf(i, cry):
                exb_v[i, pl.ds(0, 16)] = ex16_v[i, :]
                for j in range(1, 8):
                    exb_v[i, pl.ds(16 * j, 16)] = zero16
                return cry
            lax.fori_loop(0, CD, cpf, 0)
            pltpu.sync_copy(exb_v, den_sp.at[idx_dl.at[0]], add=True)
            return carry
        lax.fori_loop(0, CHD, chunk1, 0)
        plsc.subcore_barrier()
        pltpu.sync_copy(den_sp.at[rows], dent.at[c, 1, rows])

    return den


def _make_num_sc(half, hd_vregs):
    @functools.partial(
        pl.kernel,
        out_type=jax.ShapeDtypeStruct((2, NHR, 128), jnp.float32),
        mesh=plsc.VectorSubcoreMesh(**_SC_MESH),
        scratch_types=[
            pltpu.VMEM((CN,), jnp.int32),
            pltpu.VMEM((1, CN), jnp.int32),
            pltpu.VMEM((1, CN), jnp.int32),
            pltpu.VMEM((CN, 16), jnp.float32),
            pltpu.VMEM((CN, 128), jnp.float32),
            pltpu.VMEM((8, 128), jnp.float32),
            pltpu.VMEM_SHARED((NHR, 128), jnp.float32),
            pltpu.SemaphoreType.DMA,
        ])
    def num(pk, exin, htab, aggh,
            pk_v, idx_s, idx_dl, ex16_v, h_v, z_v, out_sp, sem):
        s = lax.axis_index("s")
        c = lax.axis_index("c")
        w = c * NT + s
        zero16 = jnp.zeros((16,), jnp.float32)
        for i in range(8):
            for j in range(8):
                z_v[i, pl.ds(16 * j, 16)] = zero16

        def zsp(k, carry):
            pltpu.sync_copy(z_v, out_sp.at[pl.ds(s * HROWS_PT + k * 8, 8)])
            return carry
        lax.fori_loop(0, HROWS_PT // 8, zsp, 0)
        plsc.subcore_barrier()

        def edge_chunk(ci, carry):
            g = w * CHN + ci
            pltpu.sync_copy(pk.at[pl.ds(g * CN, CN)], pk_v)
            for t in range(CN // 16):
                p = pk_v[pl.ds(16 * t, 16)]
                dl = (p >> 15) - half * NH
                ok = (dl >= 0) & (dl < NH)
                idx_s[0, pl.ds(16 * t, 16)] = p & 32767
                idx_dl[0, pl.ds(16 * t, 16)] = jnp.where(ok, dl, NHR - 1)
            cps = [pltpu.async_copy(exin.at[pl.ds(g * CN, CN)], ex16_v, sem),
                   pltpu.async_copy(htab.at[idx_s.at[0]], h_v, sem)]
            for cp in cps:
                cp.wait()

            def mulf(i, cry):
                av = ex16_v[i, :]
                for j in range(8):
                    a = av[j // hd_vregs]
                    h_v[i, pl.ds(16 * j, 16)] = h_v[i, pl.ds(16 * j, 16)] * a
                return cry
            lax.fori_loop(0, CN, mulf, 0)
            pltpu.sync_copy(h_v, out_sp.at[idx_dl.at[0]], add=True)
            return carry
        lax.fori_loop(0, CHN, edge_chunk, 0)
        plsc.subcore_barrier()
        rows = pl.ds(s * HROWS_PT, HROWS_PT)
        pltpu.sync_copy(out_sp.at[rows], aggh.at[c, rows])

    return num


_den_sc = _make_den_sc()
_num_sc = {(h, hd): _make_num_sc(h, hd)
           for h in (0, 1) for hd in (HEAD_DIM // 16, HID // 16)}


def _gat_layer_sc(pk, astab, adtab, htab, hd_vregs):
    ex, dent = _den_sc(pk, astab, adtab)
    lo = _num_sc[(0, hd_vregs)](pk, ex, htab)
    hi = _num_sc[(1, hd_vregs)](pk, ex, htab)
    agg = jnp.concatenate([lo[:, :NH], hi[:, :NH]], axis=1)
    den = jnp.concatenate([dent[:, 0, :NH, 0:16], dent[:, 1, :NH, 0:16]],
                          axis=1)
    return agg, den


def _edge_prep(ei):
    loops = jnp.arange(N, dtype=jnp.int32)
    src = jnp.concatenate([ei[0], loops])
    dst = jnp.concatenate([ei[1], loops])
    src = jnp.pad(src, (0, EP_PAD - EP), constant_values=N)
    dst = jnp.pad(dst, (0, EP_PAD - EP), constant_values=N)
    return src, dst


def _fold_att(W, att):
    heads, hd = att.shape
    Wr = W.reshape(W.shape[0], heads, hd)
    return jnp.einsum('ihd,hd->ih', Wr, att)


def kernel(x, form_edge_index, role_edge_index,
           W1f, as1f, ad1f, b1f, W2f, as2f, ad2f, b2f,
           W1r, as1r, ad1r, b1r, W2r, as2r, ad2r, b2r,
           Wfu1, bfu1, Wfu2, bfu2, gamma, beta):
    f32 = jnp.float32
    fs, fd = _edge_prep(form_edge_index)
    rs, rd = _edge_prep(role_edge_index)
    pk_f = fs | (fd << 15)
    pk_r = rs | (rd << 15)

    xp = jnp.pad(x, ((0, N_PAD - N), (0, 0)))
    att_cols = jnp.concatenate(
        [_fold_att(W1f, as1f), _fold_att(W1f, ad1f),
         _fold_att(W1r, as1r), _fold_att(W1r, ad1r)], axis=1)
    wcat = jnp.concatenate(
        [W1f, W1r, jnp.pad(att_cols, ((0, 0), (0, 112)))], axis=1)
    H = _mm(xp, wcat)

    p128 = lambda a: jnp.pad(a[:N_ROWS], ((0, 0), (0, 128 - a.shape[1])))
    pn = lambda a: jnp.pad(a, ((0, 0), (0, N_PAD - N_ROWS), (0, 0)))
    a1f, d1f = _gat_layer_sc(pk_f, p128(H[:, 256:260]), p128(H[:, 260:264]),
                             H[:N_ROWS, 0:128], HEAD_DIM // 16)
    a1r, d1r = _gat_layer_sc(pk_r, p128(H[:, 264:268]), p128(H[:, 268:272]),
                             H[:N_ROWS, 128:256], HEAD_DIM // 16)
    agg1 = jnp.stack([pn(a1f), pn(a1r)])
    den1 = jnp.stack([pn(d1f), pn(d1r)])

    b1 = jnp.stack([b1f, b1r])
    w2cat = jnp.stack([
        jnp.concatenate([W2f, _fold_att(W2f, as2f), _fold_att(W2f, ad2f),
                         jnp.zeros((128, 126), f32)], axis=1),
        jnp.concatenate([W2r, _fold_att(W2r, as2r), _fold_att(W2r, ad2r),
                         jnp.zeros((128, 126), f32)], axis=1)])
    H2 = _stageb(agg1, den1, b1, w2cat)

    a2f, d2f = _gat_layer_sc(pk_f, p128(H2[0, :, 128:129]),
                             p128(H2[0, :, 129:130]),
                             H2[0, :N_ROWS, 0:128], HID // 16)
    a2r, d2r = _gat_layer_sc(pk_r, p128(H2[1, :, 128:129]),
                             p128(H2[1, :, 129:130]),
                             H2[1, :N_ROWS, 0:128], HID // 16)

    out = _stagec(pn(a2f), pn(a2r), pn(d2f), pn(d2r),
                  b2f, b2r, Wfu1, bfu1, Wfu2, bfu2, gamma, beta)
    return out[:N]
